# baseline (speedup 1.0000x reference)
import functools

import jax
import jax.numpy as jnp
from jax import lax
from jax.experimental import pallas as pl
from jax.experimental.pallas import tpu as pltpu
from jax.experimental.pallas import tpu_sc as plsc

_B = 16384
_D = 128
_ND = 32
_NC = 2
_NS = 16
_NW = _NC * _NS
_RPW = _B // _NW
_HPW = _RPW // 2
_NPH = 8
_PPW = _RPW // _NPH
_NPPW = _PPW * _ND // _D

_NROWS = _B * _ND // _D
_NBLK = 4
_NRB = _NROWS // _NBLK
_NHPW = _HPW * _ND // _D

_ROTS = ((13, 15, 26, 6), (17, 29, 16, 24))
_KS = (0x0, 0x1, 0x1BD11BDB)


def _rotl(x, r):
    return lax.shift_left(x, jnp.uint32(r)) | lax.shift_right_logical(
        x, jnp.uint32(32 - r))


def _threefry_noise(c2):
    x0 = c2 ^ c2
    x1 = c2 + jnp.uint32(_KS[1])
    for i in range(5):
        for r in _ROTS[i % 2]:
            x0 = x0 + x1
            x1 = _rotl(x1, r)
            x1 = x0 ^ x1
        x0 = x0 + jnp.uint32(_KS[(i + 1) % 3])
        x1 = x1 + jnp.uint32((_KS[(i + 2) % 3] + i + 1) & 0xFFFFFFFF)
    bits = x0 ^ x1
    mant = lax.convert_element_type(
        lax.shift_right_logical(bits, jnp.uint32(9)), jnp.int32)
    return lax.convert_element_type(mant, jnp.float32) * jnp.float32(
        1.0 / 8388608.0) - jnp.float32(0.5)


def _tc_noise_body(inv_ref, o_ref):
    blk = pl.program_id(0)
    base = blk * (_NRB * _D)
    flat = (lax.broadcasted_iota(jnp.int32, (_NRB, _D), 0) * _D
            + lax.broadcasted_iota(jnp.int32, (_NRB, _D), 1)) + base
    c2 = lax.convert_element_type(flat, jnp.uint32)
    o_ref[...] = _threefry_noise(c2) * inv_ref[...]


@functools.lru_cache(maxsize=1)
def _tc_noise_call():
    return pl.pallas_call(
        _tc_noise_body,
        grid=(_NBLK,),
        in_specs=[pl.BlockSpec((1, _D), lambda i: (0, 0))],
        out_specs=pl.BlockSpec((_NRB, _D), lambda i: (i, 0)),
        out_shape=jax.ShapeDtypeStruct((_NROWS, _D), jnp.float32),
    )


def _sc_body(in_hbm, idx_hbm, noise_hbm, out_hbm,
             buf0, buf1, buf2, nbuf0, nbuf1, nbuf2, idx_v,
             si0, si1, si2, sn0, sn1, sn2, so):
    c = lax.axis_index("c")
    s = lax.axis_index("s")
    wid = s * _NC + c
    row0 = wid * _RPW

    bufs = (buf0, buf1, buf2)
    nbufs = (nbuf0, nbuf1, nbuf2)
    sis = (si0, si1, si2)
    sns = (sn0, sn1, sn2)

    def start_in(p):
        r0 = row0 + p * _PPW
        nr0 = pl.multiple_of(wid * (_NPH * _NPPW) + p * _NPPW, 8)
        ci = pltpu.async_copy(in_hbm.at[pl.ds(r0, _PPW)], bufs[p % 3],
                              sis[p % 3])
        cn = pltpu.async_copy(noise_hbm.at[pl.ds(nr0, _NPPW)], nbufs[p % 3],
                              sns[p % 3])
        return ci, cn

    copies = [start_in(0), start_in(1), start_in(2)]
    pltpu.sync_copy(idx_hbm, idx_v)
    lane_i = lax.iota(jnp.int32, 16)
    cols = [idx_v[pl.ds(0, 16)], idx_v[pl.ds(16, 16)]]

    copies_out = []
    drained = []
    for p in range(_NPH):
        b = p % 3
        ci, cn = copies[p]
        ci.wait()
        cn.wait()
        buf, nbuf = bufs[b], nbufs[b]

        def row_step(t, carry, buf=buf, nbuf=nbuf):
            rv = lane_i * 0 + t * 4
            for u in range(4):
                row_vec = rv + u
                for h in range(2):
                    n = nbuf[t, pl.ds(u * _ND + 16 * h, 16)]
                    x = plsc.load_gather(buf, [row_vec, cols[h]])
                    plsc.store_scatter(buf, [row_vec, cols[h]], x + n)
            return carry

        lax.fori_loop(0, _PPW // 4, row_step, 0)
        r0 = row0 + p * _PPW
        copies_out.append(
            pltpu.async_copy(buf, out_hbm.at[pl.ds(r0, _PPW)], so))
        q = p + 2
        if 3 <= q < _NPH:
            drained.append(q - 3)
            copies_out[q - 3].wait()
            copies.append(start_in(q))
    for p in range(_NPH):
        if p not in drained:
            copies_out[p].wait()


@functools.lru_cache(maxsize=1)
def _sc_call():
    return pl.kernel(
        _sc_body,
        out_type=jax.ShapeDtypeStruct((_B, _D), jnp.float32),
        mesh=plsc.VectorSubcoreMesh(core_axis_name="c", subcore_axis_name="s",
                                    num_cores=_NC, num_subcores=_NS),
        compiler_params=pltpu.CompilerParams(needs_layout_passes=False),
        scratch_types=[
            pltpu.VMEM((_PPW, _D), jnp.float32),
            pltpu.VMEM((_PPW, _D), jnp.float32),
            pltpu.VMEM((_PPW, _D), jnp.float32),
            pltpu.VMEM((_NPPW, _D), jnp.float32),
            pltpu.VMEM((_NPPW, _D), jnp.float32),
            pltpu.VMEM((_NPPW, _D), jnp.float32),
            pltpu.VMEM((_ND,), jnp.int32),
            pltpu.SemaphoreType.DMA,
            pltpu.SemaphoreType.DMA,
            pltpu.SemaphoreType.DMA,
            pltpu.SemaphoreType.DMA,
            pltpu.SemaphoreType.DMA,
            pltpu.SemaphoreType.DMA,
            pltpu.SemaphoreType.DMA,
        ],
    )


def kernel(inputs, discrete_shift, discrete_scale, discrete_idx):
    inv_tiled = jnp.tile(jnp.float32(1.0) / discrete_scale, _D // _ND)
    noise = _tc_noise_call()(inv_tiled.reshape(1, _D))
    return _sc_call()(inputs, discrete_idx, noise)


# speedup vs baseline: 1.0270x; 1.0270x over previous
import functools

import jax
import jax.numpy as jnp
from jax import lax
from jax.experimental import pallas as pl
from jax.experimental.pallas import tpu as pltpu
from jax.experimental.pallas import tpu_sc as plsc

_B = 16384
_D = 128
_ND = 32
_NC = 2
_NS = 16
_NW = _NC * _NS
_RPW = _B // _NW
_HPW = _RPW // 2
_NPH = 4
_PPW = _RPW // _NPH
_NPPW = _PPW * _ND // _D

_NROWS = _B * _ND // _D
_NBLK = 4
_NRB = _NROWS // _NBLK
_NHPW = _HPW * _ND // _D

_ROTS = ((13, 15, 26, 6), (17, 29, 16, 24))
_KS = (0x0, 0x1, 0x1BD11BDB)


def _rotl(x, r):
    return lax.shift_left(x, jnp.uint32(r)) | lax.shift_right_logical(
        x, jnp.uint32(32 - r))


def _threefry_noise(c2):
    x0 = c2 ^ c2
    x1 = c2 + jnp.uint32(_KS[1])
    for i in range(5):
        for r in _ROTS[i % 2]:
            x0 = x0 + x1
            x1 = _rotl(x1, r)
            x1 = x0 ^ x1
        x0 = x0 + jnp.uint32(_KS[(i + 1) % 3])
        x1 = x1 + jnp.uint32((_KS[(i + 2) % 3] + i + 1) & 0xFFFFFFFF)
    bits = x0 ^ x1
    mant = lax.convert_element_type(
        lax.shift_right_logical(bits, jnp.uint32(9)), jnp.int32)
    return lax.convert_element_type(mant, jnp.float32) * jnp.float32(
        1.0 / 8388608.0) - jnp.float32(0.5)


def _tc_noise_body(inv_ref, o_ref):
    blk = pl.program_id(0)
    base = blk * (_NRB * _D)
    flat = (lax.broadcasted_iota(jnp.int32, (_NRB, _D), 0) * _D
            + lax.broadcasted_iota(jnp.int32, (_NRB, _D), 1)) + base
    c2 = lax.convert_element_type(flat, jnp.uint32)
    o_ref[...] = _threefry_noise(c2) * inv_ref[...]


@functools.lru_cache(maxsize=1)
def _tc_noise_call():
    return pl.pallas_call(
        _tc_noise_body,
        grid=(_NBLK,),
        in_specs=[pl.BlockSpec((1, _D), lambda i: (0, 0))],
        out_specs=pl.BlockSpec((_NRB, _D), lambda i: (i, 0)),
        out_shape=jax.ShapeDtypeStruct((_NROWS, _D), jnp.float32),
    )


def _sc_body(in_hbm, idx_hbm, noise_hbm, out_hbm,
             buf0, buf1, buf2, nbuf0, nbuf1, nbuf2, idx_v,
             si0, si1, si2, sn0, sn1, sn2, so):
    c = lax.axis_index("c")
    s = lax.axis_index("s")
    wid = s * _NC + c
    row0 = wid * _RPW

    bufs = (buf0, buf1, buf2)
    nbufs = (nbuf0, nbuf1, nbuf2)
    sis = (si0, si1, si2)
    sns = (sn0, sn1, sn2)

    def start_in(p):
        r0 = row0 + p * _PPW
        nr0 = pl.multiple_of(wid * (_NPH * _NPPW) + p * _NPPW, 8)
        ci = pltpu.async_copy(in_hbm.at[pl.ds(r0, _PPW)], bufs[p % 3],
                              sis[p % 3])
        cn = pltpu.async_copy(noise_hbm.at[pl.ds(nr0, _NPPW)], nbufs[p % 3],
                              sns[p % 3])
        return ci, cn

    copies = [start_in(0), start_in(1), start_in(2)]
    pltpu.sync_copy(idx_hbm, idx_v)
    lane_i = lax.iota(jnp.int32, 16)
    cols = [idx_v[pl.ds(0, 16)], idx_v[pl.ds(16, 16)]]

    copies_out = []
    drained = []
    for p in range(_NPH):
        b = p % 3
        ci, cn = copies[p]
        ci.wait()
        cn.wait()
        buf, nbuf = bufs[b], nbufs[b]

        def row_step(t, carry, buf=buf, nbuf=nbuf):
            rv = lane_i * 0 + t * 4
            for u in range(4):
                row_vec = rv + u
                for h in range(2):
                    n = nbuf[t, pl.ds(u * _ND + 16 * h, 16)]
                    x = plsc.load_gather(buf, [row_vec, cols[h]])
                    plsc.store_scatter(buf, [row_vec, cols[h]], x + n)
            return carry

        lax.fori_loop(0, _PPW // 4, row_step, 0)
        r0 = row0 + p * _PPW
        copies_out.append(
            pltpu.async_copy(buf, out_hbm.at[pl.ds(r0, _PPW)], so))
        q = p + 2
        if 3 <= q < _NPH:
            drained.append(q - 3)
            copies_out[q - 3].wait()
            copies.append(start_in(q))
    for p in range(_NPH):
        if p not in drained:
            copies_out[p].wait()


@functools.lru_cache(maxsize=1)
def _sc_call():
    return pl.kernel(
        _sc_body,
        out_type=jax.ShapeDtypeStruct((_B, _D), jnp.float32),
        mesh=plsc.VectorSubcoreMesh(core_axis_name="c", subcore_axis_name="s",
                                    num_cores=_NC, num_subcores=_NS),
        compiler_params=pltpu.CompilerParams(needs_layout_passes=False),
        scratch_types=[
            pltpu.VMEM((_PPW, _D), jnp.float32),
            pltpu.VMEM((_PPW, _D), jnp.float32),
            pltpu.VMEM((_PPW, _D), jnp.float32),
            pltpu.VMEM((_NPPW, _D), jnp.float32),
            pltpu.VMEM((_NPPW, _D), jnp.float32),
            pltpu.VMEM((_NPPW, _D), jnp.float32),
            pltpu.VMEM((_ND,), jnp.int32),
            pltpu.SemaphoreType.DMA,
            pltpu.SemaphoreType.DMA,
            pltpu.SemaphoreType.DMA,
            pltpu.SemaphoreType.DMA,
            pltpu.SemaphoreType.DMA,
            pltpu.SemaphoreType.DMA,
            pltpu.SemaphoreType.DMA,
        ],
    )


def kernel(inputs, discrete_shift, discrete_scale, discrete_idx):
    inv_tiled = jnp.tile(jnp.float32(1.0) / discrete_scale, _D // _ND)
    noise = _tc_noise_call()(inv_tiled.reshape(1, _D))
    return _sc_call()(inputs, discrete_idx, noise)


# speedup vs baseline: 1.0285x; 1.0015x over previous
import functools

import jax
import jax.numpy as jnp
from jax import lax
from jax.experimental import pallas as pl
from jax.experimental.pallas import tpu as pltpu
from jax.experimental.pallas import tpu_sc as plsc

_B = 16384
_D = 128
_ND = 32
_NC = 2
_NS = 16
_NW = _NC * _NS
_RPW = _B // _NW
_NPH = 4
_PPW = _RPW // _NPH
_NPPW = _PPW * _ND // _D

_NROWS = _B * _ND // _D
_NBLK = 4
_NRB = _NROWS // _NBLK

_ROTS = ((13, 15, 26, 6), (17, 29, 16, 24))
_KS = (0x0, 0x1, 0x1BD11BDB)


def _rotl(x, r):
    return lax.shift_left(x, jnp.uint32(r)) | lax.shift_right_logical(
        x, jnp.uint32(32 - r))


def _threefry_noise(c2):
    x0 = c2 ^ c2
    x1 = c2 + jnp.uint32(_KS[1])
    for i in range(5):
        for r in _ROTS[i % 2]:
            x0 = x0 + x1
            x1 = _rotl(x1, r)
            x1 = x0 ^ x1
        x0 = x0 + jnp.uint32(_KS[(i + 1) % 3])
        x1 = x1 + jnp.uint32((_KS[(i + 2) % 3] + i + 1) & 0xFFFFFFFF)
    bits = x0 ^ x1
    mant = lax.convert_element_type(
        lax.shift_right_logical(bits, jnp.uint32(9)), jnp.int32)
    return lax.convert_element_type(mant, jnp.float32) * jnp.float32(
        1.0 / 8388608.0) - jnp.float32(0.5)


def _tc_noise_body(inv_ref, o_ref):
    blk = pl.program_id(0)
    base = blk * (_NRB * _D)
    flat = (lax.broadcasted_iota(jnp.int32, (_NRB, _D), 0) * _D
            + lax.broadcasted_iota(jnp.int32, (_NRB, _D), 1)) + base
    c2 = lax.convert_element_type(flat, jnp.uint32)
    o_ref[...] = _threefry_noise(c2) * inv_ref[...]


@functools.lru_cache(maxsize=1)
def _tc_noise_call():
    return pl.pallas_call(
        _tc_noise_body,
        grid=(_NBLK,),
        in_specs=[pl.BlockSpec((1, _D), lambda i: (0, 0))],
        out_specs=pl.BlockSpec((_NRB, _D), lambda i: (i, 0)),
        out_shape=jax.ShapeDtypeStruct((_NROWS, _D), jnp.float32),
    )


def _sc_body(in_hbm, idx_hbm, noise_hbm, out_hbm,
             buf0, buf1, buf2, nbuf0, nbuf1, nbuf2, idx_v,
             si0, si1, si2, sn0, sn1, sn2, so):
    c = lax.axis_index("c")
    s = lax.axis_index("s")
    wid = s * _NC + c
    row0 = wid * _RPW

    bufs = (buf0, buf1, buf2)
    nbufs = (nbuf0, nbuf1, nbuf2)
    sis = (si0, si1, si2)
    sns = (sn0, sn1, sn2)

    def start_in(p):
        r0 = row0 + p * _PPW
        nr0 = pl.multiple_of(wid * (_NPH * _NPPW) + p * _NPPW, 8)
        ci = pltpu.async_copy(in_hbm.at[pl.ds(r0, _PPW)], bufs[p % 3],
                              sis[p % 3])
        cn = pltpu.async_copy(noise_hbm.at[pl.ds(nr0, _NPPW)], nbufs[p % 3],
                              sns[p % 3])
        return ci, cn

    copies = [start_in(0), start_in(1), start_in(2)]
    pltpu.sync_copy(idx_hbm, idx_v)
    lane_i = lax.iota(jnp.int32, 16)
    cols = [idx_v[pl.ds(0, 16)], idx_v[pl.ds(16, 16)]]

    copies_out = []
    drained = []
    for p in range(_NPH):
        b = p % 3
        ci, cn = copies[p]
        ci.wait()
        cn.wait()
        buf, nbuf = bufs[b], nbufs[b]

        def row_step(t, carry, buf=buf, nbuf=nbuf):
            rv = lane_i * 0 + t * 4
            for u in range(4):
                row_vec = rv + u
                for h in range(2):
                    n = nbuf[t, pl.ds(u * _ND + 16 * h, 16)]
                    x = plsc.load_gather(buf, [row_vec, cols[h]])
                    plsc.store_scatter(buf, [row_vec, cols[h]], x + n)
            return carry

        lax.fori_loop(0, _PPW // 4, row_step, 0)
        r0 = row0 + p * _PPW
        copies_out.append(
            pltpu.async_copy(buf, out_hbm.at[pl.ds(r0, _PPW)], so))
        q = p + 2
        if 3 <= q < _NPH:
            drained.append(q - 3)
            copies_out[q - 3].wait()
            copies.append(start_in(q))
    for p in range(_NPH):
        if p not in drained:
            copies_out[p].wait()


@functools.lru_cache(maxsize=1)
def _sc_call():
    return pl.kernel(
        _sc_body,
        out_type=jax.ShapeDtypeStruct((_B, _D), jnp.float32),
        mesh=plsc.VectorSubcoreMesh(core_axis_name="c", subcore_axis_name="s",
                                    num_cores=_NC, num_subcores=_NS),
        compiler_params=pltpu.CompilerParams(needs_layout_passes=False),
        scratch_types=[
            pltpu.VMEM((_PPW, _D), jnp.float32),
            pltpu.VMEM((_PPW, _D), jnp.float32),
            pltpu.VMEM((_PPW, _D), jnp.float32),
            pltpu.VMEM((_NPPW, _D), jnp.float32),
            pltpu.VMEM((_NPPW, _D), jnp.float32),
            pltpu.VMEM((_NPPW, _D), jnp.float32),
            pltpu.VMEM((_ND,), jnp.int32),
            pltpu.SemaphoreType.DMA,
            pltpu.SemaphoreType.DMA,
            pltpu.SemaphoreType.DMA,
            pltpu.SemaphoreType.DMA,
            pltpu.SemaphoreType.DMA,
            pltpu.SemaphoreType.DMA,
            pltpu.SemaphoreType.DMA,
        ],
    )


def kernel(inputs, discrete_shift, discrete_scale, discrete_idx):
    inv_tiled = jnp.tile(jnp.float32(1.0) / discrete_scale, _D // _ND)
    noise = _tc_noise_call()(inv_tiled.reshape(1, _D))
    return _sc_call()(inputs, discrete_idx, noise)
